# Initial kernel scaffold; baseline (speedup 1.0000x reference)
import functools

import jax
import jax.numpy as jnp
from jax import lax
from jax.experimental import pallas as pl
from jax.experimental.pallas import tpu as pltpu
from jax.experimental.pallas import tpu_sc as plsc

N = 10000
E = 320000
D = 128

NC = 2
NS = 16
NW = NC * NS
EDGES_PER_W = E // NW
CHUNK = 80
NCHUNK = EDGES_PER_W // CHUNK
ROWS_PER_TILE = N // NS
DEGW = 16

_mesh = plsc.VectorSubcoreMesh(core_axis_name="c", subcore_axis_name="s")


def _make_sc_agg(with_deg):
  out_type = [jax.ShapeDtypeStruct((NC, N, D), jnp.float32)]
  if with_deg:
    out_type.append(jax.ShapeDtypeStruct((NC, N, DEGW), jnp.float32))
  scratch = [
      pltpu.VMEM((CHUNK,), jnp.int32),
      pltpu.VMEM((CHUNK,), jnp.int32),
      pltpu.VMEM((CHUNK, D), jnp.float32),
      pltpu.VMEM_SHARED((N, D), jnp.float32),
      pltpu.SemaphoreType.DMA,
  ]
  if with_deg:
    scratch.insert(3, pltpu.VMEM((CHUNK, DEGW), jnp.float32))
    scratch.insert(5, pltpu.VMEM_SHARED((N, DEGW), jnp.float32))

  @functools.partial(
      pl.kernel,
      out_type=tuple(out_type),
      mesh=_mesh,
      scratch_types=tuple(scratch),
  )
  def sc_agg(*refs):
    if with_deg:
      (h_hbm, src_hbm, dst_hbm, z_rows, z_deg, ones_hbm,
       sums_out, deg_out,
       idx_s, idx_d, rows_v, ones_v, sums_sh, deg_sh, sem) = refs
    else:
      (h_hbm, src_hbm, dst_hbm, z_rows,
       sums_out,
       idx_s, idx_d, rows_v, sums_sh, sem) = refs

    c = lax.axis_index("c")
    s = lax.axis_index("s")
    wid = s * NC + c
    base = wid * EDGES_PER_W
    row0 = s * ROWS_PER_TILE

    pltpu.sync_copy(z_rows.at[pl.ds(row0, ROWS_PER_TILE)],
                    sums_sh.at[pl.ds(row0, ROWS_PER_TILE)])
    if with_deg:
      pltpu.sync_copy(z_deg.at[pl.ds(row0, ROWS_PER_TILE)],
                      deg_sh.at[pl.ds(row0, ROWS_PER_TILE)])
      pltpu.sync_copy(ones_hbm, ones_v)
    plsc.subcore_barrier()

    def body(j, carry):
      off = base + j * CHUNK
      pltpu.sync_copy(src_hbm.at[pl.ds(off, CHUNK)], idx_s)
      pltpu.sync_copy(dst_hbm.at[pl.ds(off, CHUNK)], idx_d)
      pltpu.async_copy(h_hbm.at[idx_s], rows_v, sem).wait()
      pltpu.sync_copy(rows_v, sums_sh.at[idx_d], add=True)
      if with_deg:
        pltpu.sync_copy(ones_v, deg_sh.at[idx_d], add=True)
      return carry

    lax.fori_loop(0, NCHUNK, body, 0)
    plsc.subcore_barrier()

    pltpu.sync_copy(sums_sh.at[pl.ds(row0, ROWS_PER_TILE)],
                    sums_out.at[c, pl.ds(row0, ROWS_PER_TILE)])
    if with_deg:
      pltpu.sync_copy(deg_sh.at[pl.ds(row0, ROWS_PER_TILE)],
                      deg_out.at[c, pl.ds(row0, ROWS_PER_TILE)])

  return sc_agg


_sc_agg_deg = _make_sc_agg(True)
_sc_agg = _make_sc_agg(False)


RB = 2000


def _dense_body(x_ref, sp_ref, dp_ref, w12_ref, w2t_ref, b12_ref,
                gamma_ref, beta_ref, out_ref, *, with_ln):
  x = x_ref[...]
  ssum = sp_ref[0] + sp_ref[1]
  deg = dp_ref[0, :, 0:1] + dp_ref[1, :, 0:1]
  mean = ssum / jnp.maximum(deg, 1.0)
  out = (jnp.dot(x, w12_ref[...], preferred_element_type=jnp.float32)
         + b12_ref[...]
         - jnp.dot(mean, w2t_ref[...], preferred_element_type=jnp.float32))
  out = jnp.where(deg > 0.0, out, x)
  if with_ln:
    h = jnp.maximum(out, 0.0)
    mu = jnp.mean(h, axis=1, keepdims=True)
    var = jnp.mean((h - mu) * (h - mu), axis=1, keepdims=True)
    out = (h - mu) * lax.rsqrt(var + 1e-5) * gamma_ref[...] + beta_ref[...]
  out_ref[...] = out


def _make_dense(with_ln):
  body = functools.partial(_dense_body, with_ln=with_ln)
  return pl.pallas_call(
      body,
      grid=(N // RB,),
      in_specs=[
          pl.BlockSpec((RB, D), lambda i: (i, 0)),
          pl.BlockSpec((NC, RB, D), lambda i: (0, i, 0)),
          pl.BlockSpec((NC, RB, DEGW), lambda i: (0, i, 0)),
          pl.BlockSpec((D, D), lambda i: (0, 0)),
          pl.BlockSpec((D, D), lambda i: (0, 0)),
          pl.BlockSpec((1, D), lambda i: (0, 0)),
          pl.BlockSpec((1, D), lambda i: (0, 0)),
          pl.BlockSpec((1, D), lambda i: (0, 0)),
      ],
      out_specs=pl.BlockSpec((RB, D), lambda i: (i, 0)),
      out_shape=jax.ShapeDtypeStruct((N, D), jnp.float32),
  )


_dense_ln = _make_dense(True)
_dense_out = _make_dense(False)


def kernel(x, edge_index, W1_0, b1_0, W2_0, b2_0, gamma, beta,
           W1_1, b1_1, W2_1, b2_1):
  src = edge_index[0].astype(jnp.int32)
  dst = edge_index[1].astype(jnp.int32)

  z_rows = jnp.zeros((N, D), jnp.float32)
  z_deg = jnp.zeros((N, DEGW), jnp.float32)
  ones = jnp.ones((CHUNK, DEGW), jnp.float32)

  w12_0 = (W1_0 + W2_0).T
  w2t_0 = W2_0.T
  b12_0 = (b1_0 + b2_0).reshape(1, D)
  w12_1 = (W1_1 + W2_1).T
  w2t_1 = W2_1.T
  b12_1 = (b1_1 + b2_1).reshape(1, D)
  gamma2 = gamma.reshape(1, D)
  beta2 = beta.reshape(1, D)

  sums0, degp = _sc_agg_deg(x, src, dst, z_rows, z_deg, ones)
  h1 = _dense_ln(x, sums0, degp, w12_0, w2t_0, b12_0, gamma2, beta2)
  sums1 = _sc_agg(h1, src, dst, z_rows)
  out = _dense_out(h1, sums1, degp, w12_1, w2t_1, b12_1, gamma2, beta2)
  return out


# speedup vs baseline: 4.8012x; 4.8012x over previous
import functools

import jax
import jax.numpy as jnp
from jax import lax
from jax.experimental import pallas as pl
from jax.experimental.pallas import tpu as pltpu
from jax.experimental.pallas import tpu_sc as plsc

N = 10000
E = 320000
D = 128

NC = 2
NS = 16
NW = NC * NS
EDGES_PER_W = E // NW
CHUNK = 80
NCHUNK = EDGES_PER_W // CHUNK
ROWS_A = 632
ROWS_LAST = N - (NS - 1) * ROWS_A

_mesh = plsc.VectorSubcoreMesh(core_axis_name="c", subcore_axis_name="s")


def _striped(s, copy_fn):
  row0 = pl.multiple_of(s * ROWS_A, 8)

  @pl.when(s < NS - 1)
  def _():
    copy_fn(row0, ROWS_A)

  @pl.when(s == NS - 1)
  def _():
    copy_fn(row0, ROWS_LAST)


@functools.partial(
    pl.kernel,
    out_type=jax.ShapeDtypeStruct((NC, N, D), jnp.float32),
    mesh=_mesh,
    scratch_types=(
        pltpu.VMEM((CHUNK,), jnp.int32),
        pltpu.VMEM((CHUNK,), jnp.int32),
        pltpu.VMEM((CHUNK, D), jnp.float32),
        pltpu.VMEM_SHARED((N, D), jnp.float32),
        pltpu.SemaphoreType.DMA,
    ),
)
def _sc_agg(h_hbm, src_hbm, dst_hbm, z_rows, sums_out,
            idx_s, idx_d, rows_v, sums_sh, sem):
  c = lax.axis_index("c")
  s = lax.axis_index("s")
  wid = s * NC + c
  base = wid * EDGES_PER_W

  _striped(s, lambda r, n: pltpu.sync_copy(z_rows.at[pl.ds(r, n)],
                                           sums_sh.at[pl.ds(r, n)]))
  plsc.subcore_barrier()

  def body(j, carry):
    off = base + j * CHUNK
    pltpu.sync_copy(src_hbm.at[pl.ds(off, CHUNK)], idx_s)
    pltpu.sync_copy(dst_hbm.at[pl.ds(off, CHUNK)], idx_d)
    pltpu.async_copy(h_hbm.at[idx_s], rows_v, sem).wait()
    pltpu.sync_copy(rows_v, sums_sh.at[idx_d], add=True)
    return carry

  lax.fori_loop(0, NCHUNK, body, 0)
  plsc.subcore_barrier()

  _striped(s, lambda r, n: pltpu.sync_copy(sums_sh.at[pl.ds(r, n)],
                                           sums_out.at[c, pl.ds(r, n)]))


@functools.partial(
    pl.kernel,
    out_type=jax.ShapeDtypeStruct((NC, N, D), jnp.float32),
    mesh=_mesh,
    scratch_types=(
        pltpu.VMEM((CHUNK,), jnp.int32),
        pltpu.VMEM((CHUNK, D), jnp.float32),
        pltpu.VMEM_SHARED((N, D), jnp.float32),
    ),
)
def _sc_deg(dst_hbm, z_rows, ones_hbm, deg_out, idx_d, ones_v, deg_sh):
  c = lax.axis_index("c")
  s = lax.axis_index("s")
  wid = s * NC + c
  base = wid * EDGES_PER_W

  _striped(s, lambda r, n: pltpu.sync_copy(z_rows.at[pl.ds(r, n)],
                                           deg_sh.at[pl.ds(r, n)]))
  pltpu.sync_copy(ones_hbm, ones_v)
  plsc.subcore_barrier()

  def body(j, carry):
    off = base + j * CHUNK
    pltpu.sync_copy(dst_hbm.at[pl.ds(off, CHUNK)], idx_d)
    pltpu.sync_copy(ones_v, deg_sh.at[idx_d], add=True)
    return carry

  lax.fori_loop(0, NCHUNK, body, 0)
  plsc.subcore_barrier()

  _striped(s, lambda r, n: pltpu.sync_copy(deg_sh.at[pl.ds(r, n)],
                                           deg_out.at[c, pl.ds(r, n)]))


RB = 2000


def _dense_body(x_ref, sp_ref, dp_ref, w12_ref, w2t_ref, b12_ref,
                gamma_ref, beta_ref, out_ref, *, with_ln):
  x = x_ref[...]
  ssum = sp_ref[0] + sp_ref[1]
  deg = dp_ref[0, :, 0:1] + dp_ref[1, :, 0:1]
  mean = ssum / jnp.maximum(deg, 1.0)
  out = (jnp.dot(x, w12_ref[...], preferred_element_type=jnp.float32)
         + b12_ref[...]
         - jnp.dot(mean, w2t_ref[...], preferred_element_type=jnp.float32))
  out = jnp.where(deg > 0.0, out, x)
  if with_ln:
    h = jnp.maximum(out, 0.0)
    mu = jnp.mean(h, axis=1, keepdims=True)
    var = jnp.mean((h - mu) * (h - mu), axis=1, keepdims=True)
    out = (h - mu) * lax.rsqrt(var + 1e-5) * gamma_ref[...] + beta_ref[...]
  out_ref[...] = out


def _make_dense(with_ln):
  body = functools.partial(_dense_body, with_ln=with_ln)
  return pl.pallas_call(
      body,
      grid=(N // RB,),
      in_specs=[
          pl.BlockSpec((RB, D), lambda i: (i, 0)),
          pl.BlockSpec((NC, RB, D), lambda i: (0, i, 0)),
          pl.BlockSpec((NC, RB, D), lambda i: (0, i, 0)),
          pl.BlockSpec((D, D), lambda i: (0, 0)),
          pl.BlockSpec((D, D), lambda i: (0, 0)),
          pl.BlockSpec((1, D), lambda i: (0, 0)),
          pl.BlockSpec((1, D), lambda i: (0, 0)),
          pl.BlockSpec((1, D), lambda i: (0, 0)),
      ],
      out_specs=pl.BlockSpec((RB, D), lambda i: (i, 0)),
      out_shape=jax.ShapeDtypeStruct((N, D), jnp.float32),
  )


_dense_ln = _make_dense(True)
_dense_out = _make_dense(False)


def kernel(x, edge_index, W1_0, b1_0, W2_0, b2_0, gamma, beta,
           W1_1, b1_1, W2_1, b2_1):
  src = edge_index[0].astype(jnp.int32)
  dst = edge_index[1].astype(jnp.int32)

  z_rows = jnp.zeros((N, D), jnp.float32)
  ones = jnp.ones((CHUNK, D), jnp.float32)

  w12_0 = (W1_0 + W2_0).T
  w2t_0 = W2_0.T
  b12_0 = (b1_0 + b2_0).reshape(1, D)
  w12_1 = (W1_1 + W2_1).T
  w2t_1 = W2_1.T
  b12_1 = (b1_1 + b2_1).reshape(1, D)
  gamma2 = gamma.reshape(1, D)
  beta2 = beta.reshape(1, D)

  degp = _sc_deg(dst, z_rows, ones)
  sums0 = _sc_agg(x, src, dst, z_rows)
  h1 = _dense_ln(x, sums0, degp, w12_0, w2t_0, b12_0, gamma2, beta2)
  sums1 = _sc_agg(h1, src, dst, z_rows)
  out = _dense_out(h1, sums1, degp, w12_1, w2t_1, b12_1, gamma2, beta2)
  return out
